# Initial kernel scaffold; baseline (speedup 1.0000x reference)
import jax
import jax.numpy as jnp
from jax.experimental import pallas as pl


def kernel(x, edge_index, edge_attr, Wn1, bn1, Wn2, bn2, We1, be1, We2, be2, aWl, aWr, aWe, aatt, ab, cWl, cWr, cWe, catt, cb):
    raise NotImplementedError("write your pallas kernel here")


# speedup vs baseline: 21.3844x; 21.3844x over previous
import jax
import jax.numpy as jnp
from jax import lax
from jax.experimental import pallas as pl
from jax.experimental.pallas import tpu as pltpu
from jax.experimental.pallas import tpu_sc as plsc

N = 10000
E = 320000
D_NODE = 128
D_EDGE = 16
ENC = 128
OUTS = 6

NC = 2
NS = 16
NW = NC * NS
EPW = E // NW
C = 1000
NCHUNK = EPW // C
ZR8 = 1000

F32 = jnp.float32


def _nodes_body(x_ref, Wn1_ref, bn1_ref, Wn2_ref, bn2_ref, Wsrc_ref, Wdst_ref,
                srcT_ref, dstT_ref):
    h = jnp.maximum(x_ref[...] @ Wn1_ref[...] + bn1_ref[...], 0.0)
    h = h @ Wn2_ref[...] + bn2_ref[...]
    srcT_ref[...] = h @ Wsrc_ref[...]
    dstT_ref[...] = h @ Wdst_ref[...]


def _node_tables(x, Wn1, bn1, Wn2, bn2, Wsrc, Wdst):
    BN = 2000
    grid = (N // BN,)
    return pl.pallas_call(
        _nodes_body,
        grid=grid,
        in_specs=[
            pl.BlockSpec((BN, D_NODE), lambda i: (i, 0)),
            pl.BlockSpec((D_NODE, ENC), lambda i: (0, 0)),
            pl.BlockSpec((1, ENC), lambda i: (0, 0)),
            pl.BlockSpec((ENC, ENC), lambda i: (0, 0)),
            pl.BlockSpec((1, ENC), lambda i: (0, 0)),
            pl.BlockSpec((ENC, 16), lambda i: (0, 0)),
            pl.BlockSpec((ENC, 16), lambda i: (0, 0)),
        ],
        out_specs=[
            pl.BlockSpec((BN, 16), lambda i: (i, 0)),
            pl.BlockSpec((BN, 16), lambda i: (i, 0)),
        ],
        out_shape=[
            jax.ShapeDtypeStruct((N, 16), F32),
            jax.ShapeDtypeStruct((N, 16), F32),
        ],
    )(x, Wn1, bn1.reshape(1, ENC), Wn2, bn2.reshape(1, ENC), Wsrc, Wdst)


def _edges_body(ea_ref, We1_ref, be1_ref, Wec_ref, bec_ref, ee_ref):
    t = jnp.maximum(ea_ref[...] @ We1_ref[...] + be1_ref[...], 0.0)
    ee_ref[...] = t @ Wec_ref[...] + bec_ref[...]


def _edge_table(ea, We1, be1, Wec, bec):
    BE = 4000
    grid = (E // BE,)
    return pl.pallas_call(
        _edges_body,
        grid=grid,
        in_specs=[
            pl.BlockSpec((BE, D_EDGE), lambda i: (i, 0)),
            pl.BlockSpec((D_EDGE, ENC), lambda i: (0, 0)),
            pl.BlockSpec((1, ENC), lambda i: (0, 0)),
            pl.BlockSpec((ENC, 16), lambda i: (0, 0)),
            pl.BlockSpec((1, 16), lambda i: (0, 0)),
        ],
        out_specs=pl.BlockSpec((BE, 16), lambda i: (i, 0)),
        out_shape=jax.ShapeDtypeStruct((E, 16), F32),
    )(ea, We1, be1.reshape(1, ENC), Wec, bec.reshape(1, 16))


def _sc_body(srcT_h, dstT_h, eeT_h, srcI_h, dstI_h, consts_h, out_h,
             si_v, di_v, sr_v, dr_v, ee_v, vals_v, cst_v, acc_s, sem, sem2):
    c = lax.axis_index("c")
    s = lax.axis_index("s")
    wid = c * NS + s

    pltpu.sync_copy(consts_h, cst_v)

    def _z(i, carry):
        vals_v[i] = jnp.zeros((16,), F32)
        return carry

    lax.fori_loop(0, ZR8, _z, 0, unroll=8)

    @pl.when(s < N // ZR8)
    def _zero_acc():
        pltpu.sync_copy(vals_v.at[pl.ds(0, ZR8)], acc_s.at[pl.ds(s * ZR8, ZR8)])

    plsc.subcore_barrier()

    attA = cst_v[0]
    attC = cst_v[1]
    mA = cst_v[2]
    e6 = cst_v[3]
    m8 = cst_v[4]
    e9 = cst_v[5]

    for k in range(NCHUNK):
        base = wid * EPW + k * C
        pltpu.sync_copy(srcI_h.at[pl.ds(base, C)], si_v)
        pltpu.sync_copy(dstI_h.at[pl.ds(base, C)], di_v)
        g1 = pltpu.async_copy(srcT_h.at[si_v], sr_v, sem)
        g2 = pltpu.async_copy(dstT_h.at[di_v], dr_v, sem2)
        pltpu.sync_copy(eeT_h.at[pl.ds(base, C)], ee_v)
        g1.wait()
        g2.wait()

        def _edge(i, carry):
            srow = sr_v[i]
            u = srow + dr_v[i] + ee_v[i]
            l = jnp.maximum(u, 0.0) + 0.2 * jnp.minimum(u, 0.0)
            la = jnp.sum(l * attA)
            lc = jnp.sum(l * attC)
            pa = jnp.exp(jnp.broadcast_to(la, (16,)))
            pc = jnp.exp(jnp.broadcast_to(lc, (16,)))
            vals_v[i] = pa * (srow * mA + e6) + pc * (srow * m8 + e9)
            return carry

        lax.fori_loop(0, C, _edge, 0, unroll=4)
        pltpu.sync_copy(vals_v, acc_s.at[di_v], add=True)

    plsc.subcore_barrier()

    @pl.when(s < N // ZR8)
    def _dump_acc():
        pltpu.sync_copy(acc_s.at[pl.ds(s * ZR8, ZR8)],
                        out_h.at[pl.ds(c * N + s * ZR8, ZR8)])


def _sc_scatter(srcT, dstT, eeT, srcI, dstI, consts):
    mesh = plsc.VectorSubcoreMesh(core_axis_name="c", subcore_axis_name="s",
                                  num_cores=NC, num_subcores=NS)
    fn = pl.kernel(
        _sc_body,
        out_type=jax.ShapeDtypeStruct((NC * N, 16), F32),
        mesh=mesh,
        compiler_params=pltpu.CompilerParams(needs_layout_passes=False,
                                             use_tc_tiling_on_sc=False),
        scratch_types=[
            pltpu.VMEM((C,), jnp.int32),
            pltpu.VMEM((C,), jnp.int32),
            pltpu.VMEM((C, 16), F32),
            pltpu.VMEM((C, 16), F32),
            pltpu.VMEM((C, 16), F32),
            pltpu.VMEM((C, 16), F32),
            pltpu.VMEM((8, 16), F32),
            pltpu.VMEM_SHARED((N, 16), F32),
            pltpu.SemaphoreType.DMA,
            pltpu.SemaphoreType.DMA,
        ],
    )
    return fn(srcT, dstT, eeT, srcI, dstI, consts)


def _finish_body(p_ref, ab_ref, cb_ref, actor_ref, value_ref):
    acc = p_ref[pl.ds(0, N), :] + p_ref[pl.ds(N, N), :]
    actor_ref[...] = acc[:, 0:6] / (acc[:, 6:7] + 1e-16) + ab_ref[...]
    critic = acc[:, 8] / (acc[:, 9] + 1e-16) + cb_ref[0, 0]
    value_ref[...] = jnp.broadcast_to(jnp.sum(critic) * (1.0 / N), (1, 128))


def _finish(partials, ab, cb):
    return pl.pallas_call(
        _finish_body,
        out_shape=[
            jax.ShapeDtypeStruct((N, OUTS), F32),
            jax.ShapeDtypeStruct((1, 128), F32),
        ],
    )(partials, ab.reshape(1, OUTS), cb.reshape(1, 1))


def kernel(x, edge_index, edge_attr, Wn1, bn1, Wn2, bn2, We1, be1, We2, be2,
           aWl, aWr, aWe, aatt, ab, cWl, cWr, cWe, catt, cb):
    z2 = jnp.zeros((ENC, 2), F32)
    z7 = jnp.zeros((ENC, 7), F32)
    Wsrc = jnp.concatenate([aWl, z2, cWl, z7], axis=1)
    Wdst = jnp.concatenate([aWr, z2, cWr, z7], axis=1)
    Wcat = jnp.concatenate([aWe, z2, cWe, z7], axis=1)
    Wec = We2 @ Wcat
    bec = be2 @ Wcat

    lanes = jnp.arange(16)
    attA = jnp.concatenate([aatt, jnp.zeros((10,), F32)])
    attC = jnp.concatenate([jnp.zeros((8,), F32), catt, jnp.zeros((7,), F32)])
    consts = jnp.stack([
        attA,
        attC,
        (lanes < 6).astype(F32),
        (lanes == 6).astype(F32),
        (lanes == 8).astype(F32),
        (lanes == 9).astype(F32),
        jnp.zeros((16,), F32),
        jnp.zeros((16,), F32),
    ])

    srcI = edge_index[0].astype(jnp.int32)
    dstI = edge_index[1].astype(jnp.int32)

    srcT, dstT = _node_tables(x, Wn1, bn1, Wn2, bn2, Wsrc, Wdst)
    eeT = _edge_table(edge_attr, We1, be1, Wec, bec)
    partials = _sc_scatter(srcT, dstT, eeT, srcI, dstI, consts)
    actor, value = _finish(partials, ab, cb)
    return actor, value[0, 0]


# speedup vs baseline: 22.9191x; 1.0718x over previous
import jax
import jax.numpy as jnp
from jax import lax
from jax.experimental import pallas as pl
from jax.experimental.pallas import tpu as pltpu
from jax.experimental.pallas import tpu_sc as plsc

N = 10000
E = 320000
D_NODE = 128
D_EDGE = 16
ENC = 128
OUTS = 6

NC = 2
NS = 16
NW = NC * NS
EPW = E // NW
C = 1000
NCHUNK = EPW // C
ZR8 = 1000

F32 = jnp.float32


def _nodes_body(x_ref, Wn1_ref, bn1_ref, Wn2_ref, bn2_ref, Wsrc_ref, Wdst_ref,
                srcT_ref, dstT_ref):
    h = jnp.maximum(x_ref[...] @ Wn1_ref[...] + bn1_ref[...], 0.0)
    h = h @ Wn2_ref[...] + bn2_ref[...]
    srcT_ref[...] = h @ Wsrc_ref[...]
    dstT_ref[...] = h @ Wdst_ref[...]


def _node_tables(x, Wn1, bn1, Wn2, bn2, Wsrc, Wdst):
    BN = 2000
    grid = (N // BN,)
    return pl.pallas_call(
        _nodes_body,
        grid=grid,
        in_specs=[
            pl.BlockSpec((BN, D_NODE), lambda i: (i, 0)),
            pl.BlockSpec((D_NODE, ENC), lambda i: (0, 0)),
            pl.BlockSpec((1, ENC), lambda i: (0, 0)),
            pl.BlockSpec((ENC, ENC), lambda i: (0, 0)),
            pl.BlockSpec((1, ENC), lambda i: (0, 0)),
            pl.BlockSpec((ENC, 16), lambda i: (0, 0)),
            pl.BlockSpec((ENC, 16), lambda i: (0, 0)),
        ],
        out_specs=[
            pl.BlockSpec((BN, 16), lambda i: (i, 0)),
            pl.BlockSpec((BN, 16), lambda i: (i, 0)),
        ],
        out_shape=[
            jax.ShapeDtypeStruct((N, 16), F32),
            jax.ShapeDtypeStruct((N, 16), F32),
        ],
    )(x, Wn1, bn1.reshape(1, ENC), Wn2, bn2.reshape(1, ENC), Wsrc, Wdst)


def _edges_body(ea_ref, We1_ref, be1_ref, Wec_ref, bec_ref, ee_ref):
    t = jnp.maximum(ea_ref[...] @ We1_ref[...] + be1_ref[...], 0.0)
    ee_ref[...] = t @ Wec_ref[...] + bec_ref[...]


def _edge_table(ea, We1, be1, Wec, bec):
    BE = 4000
    grid = (E // BE,)
    return pl.pallas_call(
        _edges_body,
        grid=grid,
        in_specs=[
            pl.BlockSpec((BE, D_EDGE), lambda i: (i, 0)),
            pl.BlockSpec((D_EDGE, ENC), lambda i: (0, 0)),
            pl.BlockSpec((1, ENC), lambda i: (0, 0)),
            pl.BlockSpec((ENC, 16), lambda i: (0, 0)),
            pl.BlockSpec((1, 16), lambda i: (0, 0)),
        ],
        out_specs=pl.BlockSpec((BE, 16), lambda i: (i, 0)),
        out_shape=jax.ShapeDtypeStruct((E, 16), F32),
    )(ea, We1, be1.reshape(1, ENC), Wec, bec.reshape(1, 16))


def _sc_body(srcT_h, dstT_h, eeT_h, srcI_h, dstI_h, consts_h, out_h,
             si0, di0, sr0, dr0, ee0, si1, di1, sr1, dr1, ee1,
             vals_v, cst_v, acc_s, semI, semG, semE):
    c = lax.axis_index("c")
    s = lax.axis_index("s")
    wid = c * NS + s

    pltpu.sync_copy(consts_h, cst_v)

    def _z(i, carry):
        vals_v[i] = jnp.zeros((16,), F32)
        return carry

    lax.fori_loop(0, ZR8, _z, 0, unroll=8)

    @pl.when(s < N // ZR8)
    def _zero_acc():
        pltpu.sync_copy(vals_v.at[pl.ds(0, ZR8)], acc_s.at[pl.ds(s * ZR8, ZR8)])

    plsc.subcore_barrier()

    attA = cst_v[0]
    cattv = cst_v[1]
    maskP = cst_v[2]
    e714 = cst_v[3]
    m67 = cst_v[4] > 0.5

    bufs = [(si0, di0, sr0, dr0, ee0), (si1, di1, sr1, dr1, ee1)]

    def issue_idx(k):
        si, di = bufs[k % 2][0], bufs[k % 2][1]
        base = wid * EPW + k * C
        c1 = pltpu.async_copy(srcI_h.at[pl.ds(base, C)], si, semI)
        c2 = pltpu.async_copy(dstI_h.at[pl.ds(base, C)], di, semI)
        return (c1, c2)

    def issue_main(k):
        si, di, sr, dr, ee = bufs[k % 2]
        base = wid * EPW + k * C
        g1 = pltpu.async_copy(srcT_h.at[si], sr, semG)
        g2 = pltpu.async_copy(dstT_h.at[di], dr, semG)
        g3 = pltpu.async_copy(eeT_h.at[pl.ds(base, C)], ee, semG)
        return (g1, g2, g3)

    idx_cps = {0: issue_idx(0)}
    for cp in idx_cps[0]:
        cp.wait()
    main_cps = {0: issue_main(0)}
    idx_cps[1] = issue_idx(1)

    for k in range(NCHUNK):
        si, di, sr, dr, ee = bufs[k % 2]
        for cp in main_cps[k]:
            cp.wait()
        if k + 1 < NCHUNK:
            for cp in idx_cps[k + 1]:
                cp.wait()
            main_cps[k + 1] = issue_main(k + 1)

        def _edge(i, carry):
            srow = sr[i]
            u = srow + dr[i] + ee[i]
            l = jnp.maximum(u, 0.0) + 0.2 * jnp.minimum(u, 0.0)
            t = plsc.cumsum(l * attA)
            m = jnp.where(m67, l * cattv, t)
            vals_v[i] = jnp.exp(m) * (srow * maskP + e714)
            return carry

        lax.fori_loop(0, C, _edge, 0, unroll=8)
        pltpu.sync_copy(vals_v, acc_s.at[di], add=True)
        if k + 2 < NCHUNK:
            idx_cps[k + 2] = issue_idx(k + 2)

    plsc.subcore_barrier()

    @pl.when(s < N // ZR8)
    def _dump_acc():
        pltpu.sync_copy(acc_s.at[pl.ds(s * ZR8, ZR8)],
                        out_h.at[pl.ds(c * N + s * ZR8, ZR8)])


def _sc_scatter(srcT, dstT, eeT, srcI, dstI, consts):
    mesh = plsc.VectorSubcoreMesh(core_axis_name="c", subcore_axis_name="s",
                                  num_cores=NC, num_subcores=NS)
    fn = pl.kernel(
        _sc_body,
        out_type=jax.ShapeDtypeStruct((NC * N, 16), F32),
        mesh=mesh,
        compiler_params=pltpu.CompilerParams(needs_layout_passes=False,
                                             use_tc_tiling_on_sc=False),
        scratch_types=[
            pltpu.VMEM((C,), jnp.int32),
            pltpu.VMEM((C,), jnp.int32),
            pltpu.VMEM((C, 16), F32),
            pltpu.VMEM((C, 16), F32),
            pltpu.VMEM((C, 16), F32),
            pltpu.VMEM((C,), jnp.int32),
            pltpu.VMEM((C,), jnp.int32),
            pltpu.VMEM((C, 16), F32),
            pltpu.VMEM((C, 16), F32),
            pltpu.VMEM((C, 16), F32),
            pltpu.VMEM((C, 16), F32),
            pltpu.VMEM((8, 16), F32),
            pltpu.VMEM_SHARED((N, 16), F32),
            pltpu.SemaphoreType.DMA,
            pltpu.SemaphoreType.DMA,
            pltpu.SemaphoreType.DMA,
        ],
    )
    return fn(srcT, dstT, eeT, srcI, dstI, consts)


def _finish_body(p_ref, ab_ref, cb_ref, actor_ref, value_ref):
    acc = p_ref[pl.ds(0, N), :] + p_ref[pl.ds(N, N), :]
    actor_ref[...] = acc[:, 8:14] / (acc[:, 14:15] + 1e-16) + ab_ref[...]
    critic = acc[:, 6] / (acc[:, 7] + 1e-16) + cb_ref[0, 0]
    value_ref[...] = jnp.broadcast_to(jnp.sum(critic) * (1.0 / N), (1, 128))


def _finish(partials, ab, cb):
    return pl.pallas_call(
        _finish_body,
        out_shape=[
            jax.ShapeDtypeStruct((N, OUTS), F32),
            jax.ShapeDtypeStruct((1, 128), F32),
        ],
    )(partials, ab.reshape(1, OUTS), cb.reshape(1, 1))


def kernel(x, edge_index, edge_attr, Wn1, bn1, Wn2, bn2, We1, be1, We2, be2,
           aWl, aWr, aWe, aatt, ab, cWl, cWr, cWe, catt, cb):
    z2 = jnp.zeros((ENC, 2), F32)
    z8 = jnp.zeros((ENC, 8), F32)
    Wsrc = jnp.concatenate([aWl, cWl, cWl, aWl, z2], axis=1)
    Wdst = jnp.concatenate([aWr, cWr, cWr, z8], axis=1)
    Wcat = jnp.concatenate([aWe, cWe, cWe, z8], axis=1)
    Wec = We2 @ Wcat
    bec = be2 @ Wcat

    lanes = jnp.arange(16)
    consts = jnp.stack([
        jnp.concatenate([aatt, jnp.zeros((10,), F32)]),
        jnp.where((lanes >= 6) & (lanes < 8), catt[0], 0.0).astype(F32),
        ((lanes == 6) | ((lanes >= 8) & (lanes < 14))).astype(F32),
        ((lanes == 7) | (lanes == 14)).astype(F32),
        ((lanes >= 6) & (lanes < 8)).astype(F32),
        jnp.zeros((16,), F32),
        jnp.zeros((16,), F32),
        jnp.zeros((16,), F32),
    ])

    srcI = edge_index[0].astype(jnp.int32)
    dstI = edge_index[1].astype(jnp.int32)

    srcT, dstT = _node_tables(x, Wn1, bn1, Wn2, bn2, Wsrc, Wdst)
    eeT = _edge_table(edge_attr, We1, be1, Wec, bec)
    partials = _sc_scatter(srcT, dstT, eeT, srcI, dstI, consts)
    actor, value = _finish(partials, ab, cb)
    return actor, value[0, 0]


# speedup vs baseline: 62.0388x; 2.7069x over previous
import jax
import jax.numpy as jnp
from jax import lax
from jax.experimental import pallas as pl
from jax.experimental.pallas import tpu as pltpu
from jax.experimental.pallas import tpu_sc as plsc

N = 10000
E = 320000
D_NODE = 128
D_EDGE = 16
ENC = 128
OUTS = 6

NC = 2
NS = 16
NW = NC * NS
EPW = E // NW
C = 800
TOT_CHUNKS = E // C
NCHUNK = -(-TOT_CHUNKS // NW)
G = C // 16
ZR8 = 1000

F32 = jnp.float32


def _nodes_body(x_ref, Wn1_ref, bn1_ref, Wn2_ref, bn2_ref, Wsrc_ref, Wdst_ref,
                srcT_ref, dstT_ref):
    h = jnp.maximum(x_ref[...] @ Wn1_ref[...] + bn1_ref[...], 0.0)
    h = h @ Wn2_ref[...] + bn2_ref[...]
    srcT_ref[...] = h @ Wsrc_ref[...]
    dstT_ref[...] = h @ Wdst_ref[...]


def _node_tables(x, Wn1, bn1, Wn2, bn2, Wsrc, Wdst):
    BN = 2000
    grid = (N // BN,)
    return pl.pallas_call(
        _nodes_body,
        grid=grid,
        in_specs=[
            pl.BlockSpec((BN, D_NODE), lambda i: (i, 0)),
            pl.BlockSpec((D_NODE, ENC), lambda i: (0, 0)),
            pl.BlockSpec((1, ENC), lambda i: (0, 0)),
            pl.BlockSpec((ENC, ENC), lambda i: (0, 0)),
            pl.BlockSpec((1, ENC), lambda i: (0, 0)),
            pl.BlockSpec((ENC, 16), lambda i: (0, 0)),
            pl.BlockSpec((ENC, 16), lambda i: (0, 0)),
        ],
        out_specs=[
            pl.BlockSpec((BN, 16), lambda i: (i, 0)),
            pl.BlockSpec((BN, 16), lambda i: (i, 0)),
        ],
        out_shape=[
            jax.ShapeDtypeStruct((N, 16), F32),
            jax.ShapeDtypeStruct((N, 16), F32),
        ],
    )(x, Wn1, bn1.reshape(1, ENC), Wn2, bn2.reshape(1, ENC), Wsrc, Wdst)


def _edges_body(eaT_ref, W1T_ref, b1c_ref, W2T_ref, b2c_ref, *out_refs):
    t = jnp.maximum(W1T_ref[...] @ eaT_ref[...] + b1c_ref[...], 0.0)
    ee = W2T_ref[...] @ t + b2c_ref[...]
    for j in range(7):
        out_refs[j][...] = ee[j].reshape(out_refs[j].shape)


def _edge_planes(eaT, W1T, b1c, W2T, b2c):
    BE = 3200
    grid = (E // BE,)
    return pl.pallas_call(
        _edges_body,
        grid=grid,
        in_specs=[
            pl.BlockSpec((D_EDGE, BE), lambda i: (0, i)),
            pl.BlockSpec((ENC, D_EDGE), lambda i: (0, 0)),
            pl.BlockSpec((ENC, 1), lambda i: (0, 0)),
            pl.BlockSpec((8, ENC), lambda i: (0, 0)),
            pl.BlockSpec((8, 1), lambda i: (0, 0)),
        ],
        out_specs=[pl.BlockSpec((1, BE // 128, 128), lambda i: (i, 0, 0))
                   for _ in range(7)],
        out_shape=[jax.ShapeDtypeStruct((E // BE, BE // 128, 128), F32)
                   for _ in range(7)],
    )(eaT, W1T, b1c, W2T, b2c)


def _sc_body(srcT_h, dstT_h, e0_h, e1_h, e2_h, e3_h, e4_h, e5_h, e6_h,
             srcI_h, dstI_h, consts_h, out_h,
             si0, di0, sr0, dr0, ee0, si1, di1, sr1, dr1, ee1,
             vals_v, cst_v, zbuf_v, acc_s, semI, semG):
    c = lax.axis_index("c")
    s = lax.axis_index("s")
    wid = c * NS + s
    ee_hs = (e0_h, e1_h, e2_h, e3_h, e4_h, e5_h, e6_h)

    pltpu.sync_copy(consts_h, cst_v)

    def _z(i, carry):
        zbuf_v[i] = jnp.zeros((16,), F32)
        return carry

    lax.fori_loop(0, 500, _z, 0, unroll=8)

    @pl.when(s < N // ZR8)
    def _zero_acc():
        pltpu.sync_copy(zbuf_v, acc_s.at[pl.ds(s * ZR8, 500)])
        pltpu.sync_copy(zbuf_v, acc_s.at[pl.ds(s * ZR8 + 500, 500)])

    plsc.subcore_barrier()

    attrow = cst_v[0]
    a_s = [attrow[j] for j in range(6)]
    catt_s = attrow[6]
    iota16 = lax.iota(jnp.int32, 16)
    col = [jnp.full((16,), j, jnp.int32) for j in range(16)]

    bufs = [(si0, di0, sr0, dr0, ee0), (si1, di1, sr1, dr1, ee1)]

    def issue_idx(k):
        si, di, _, _, ee = bufs[k % 2]
        cid = k * NW + wid
        cps = [pltpu.async_copy(srcI_h.at[cid], si, semI),
               pltpu.async_copy(dstI_h.at[cid], di, semI)]
        for j in range(7):
            cps.append(pltpu.async_copy(ee_hs[j].at[cid],
                                        ee.at[pl.ds(j * C, C)], semI))
        return cps

    def issue_main(k):
        si, di, sr, dr, _ = bufs[k % 2]
        return [pltpu.async_copy(srcT_h.at[si], sr, semG),
                pltpu.async_copy(dstT_h.at[di], dr, semG)]

    def compute_chunk(si, di, sr, dr, ee):
        def _group(g, carry):
            ridx = iota16 + g * 16
            sv = [plsc.load_gather(sr, [ridx, col[j]]) for j in range(7)]
            dv = [plsc.load_gather(dr, [ridx, col[j]]) for j in range(7)]
            l = []
            for j in range(7):
                u = sv[j] + dv[j] + ee[pl.ds(j * C + g * 16, 16)]
                l.append(jnp.maximum(u, 0.0) + 0.2 * jnp.minimum(u, 0.0))
            la = ((l[0] * a_s[0] + l[1] * a_s[1])
                  + (l[2] * a_s[2] + l[3] * a_s[3])
                  + (l[4] * a_s[4] + l[5] * a_s[5]))
            pa = jnp.exp(la)
            pc = jnp.exp(l[6] * catt_s)
            for j in range(6):
                plsc.store_scatter(vals_v, [ridx, col[8 + j]], pa * sv[j])
            plsc.store_scatter(vals_v, [ridx, col[14]], pa)
            plsc.store_scatter(vals_v, [ridx, col[6]], pc * sv[6])
            plsc.store_scatter(vals_v, [ridx, col[7]], pc)
            return carry

        lax.fori_loop(0, G, _group, 0, unroll=2)
        pltpu.sync_copy(vals_v, acc_s.at[di], add=True)

    NSTAT = TOT_CHUNKS // NW
    idx_cps = {0: issue_idx(0)}
    for cp in idx_cps[0]:
        cp.wait()
    main_cps = {0: issue_main(0)}
    idx_cps[1] = issue_idx(1)

    for k in range(NSTAT):
        si, di, sr, dr, ee = bufs[k % 2]
        for cp in main_cps[k]:
            cp.wait()
        if k + 1 < NSTAT:
            for cp in idx_cps[k + 1]:
                cp.wait()
            main_cps[k + 1] = issue_main(k + 1)
        compute_chunk(si, di, sr, dr, ee)
        if k + 2 < NSTAT:
            idx_cps[k + 2] = issue_idx(k + 2)

    TAIL = TOT_CHUNKS - NSTAT * NW
    if TAIL:
        @pl.when(wid < TAIL)
        def _tail():
            si, di, sr, dr, ee = bufs[NSTAT % 2]
            for cp in issue_idx(NSTAT):
                cp.wait()
            for cp in issue_main(NSTAT):
                cp.wait()
            compute_chunk(si, di, sr, dr, ee)

    plsc.subcore_barrier()

    @pl.when(s < N // ZR8)
    def _dump_acc():
        pltpu.sync_copy(acc_s.at[pl.ds(s * ZR8, ZR8)],
                        out_h.at[pl.ds(c * N + s * ZR8, ZR8)])


def _sc_scatter(srcT, dstT, eeP, srcI, dstI, consts):
    mesh = plsc.VectorSubcoreMesh(core_axis_name="c", subcore_axis_name="s",
                                  num_cores=NC, num_subcores=NS)
    fn = pl.kernel(
        _sc_body,
        out_type=jax.ShapeDtypeStruct((NC * N, 16), F32),
        mesh=mesh,
        compiler_params=pltpu.CompilerParams(needs_layout_passes=False,
                                             use_tc_tiling_on_sc=False),
        scratch_types=[
            pltpu.VMEM((C,), jnp.int32),
            pltpu.VMEM((C,), jnp.int32),
            pltpu.VMEM((C, 16), F32),
            pltpu.VMEM((C, 16), F32),
            pltpu.VMEM((7 * C,), F32),
            pltpu.VMEM((C,), jnp.int32),
            pltpu.VMEM((C,), jnp.int32),
            pltpu.VMEM((C, 16), F32),
            pltpu.VMEM((C, 16), F32),
            pltpu.VMEM((7 * C,), F32),
            pltpu.VMEM((C, 16), F32),
            pltpu.VMEM((8, 16), F32),
            pltpu.VMEM((500, 16), F32),
            pltpu.VMEM_SHARED((N, 16), F32),
            pltpu.SemaphoreType.DMA,
            pltpu.SemaphoreType.DMA,
        ],
    )
    return fn(srcT, dstT, *eeP, srcI, dstI, consts)


def _finish_body(p_ref, ab_ref, cb_ref, actor_ref, value_ref):
    acc = p_ref[pl.ds(0, N), :] + p_ref[pl.ds(N, N), :]
    actor_ref[...] = acc[:, 8:14] / (acc[:, 14:15] + 1e-16) + ab_ref[...]
    critic = acc[:, 6] / (acc[:, 7] + 1e-16) + cb_ref[0, 0]
    value_ref[...] = jnp.broadcast_to(jnp.sum(critic) * (1.0 / N), (1, 128))


def _finish(partials, ab, cb):
    return pl.pallas_call(
        _finish_body,
        out_shape=[
            jax.ShapeDtypeStruct((N, OUTS), F32),
            jax.ShapeDtypeStruct((1, 128), F32),
        ],
    )(partials, ab.reshape(1, OUTS), cb.reshape(1, 1))


def kernel(x, edge_index, edge_attr, Wn1, bn1, Wn2, bn2, We1, be1, We2, be2,
           aWl, aWr, aWe, aatt, ab, cWl, cWr, cWe, catt, cb):
    z9 = jnp.zeros((ENC, 9), F32)
    Wsrc = jnp.concatenate([aWl, cWl, z9], axis=1)
    Wdst = jnp.concatenate([aWr, cWr, z9], axis=1)
    Wcat = jnp.concatenate([aWe, cWe, jnp.zeros((ENC, 1), F32)], axis=1)
    Wec = jnp.dot(We2, Wcat, precision=lax.Precision.HIGHEST)
    bec = jnp.dot(be2, Wcat, precision=lax.Precision.HIGHEST)

    consts = jnp.zeros((8, 16), F32)
    consts = consts.at[0, 0:6].set(aatt)
    consts = consts.at[0, 6].set(catt[0])

    srcI = edge_index[0].astype(jnp.int32)
    dstI = edge_index[1].astype(jnp.int32)

    srcT, dstT = _node_tables(x, Wn1, bn1, Wn2, bn2, Wsrc, Wdst)
    eeP = _edge_planes(edge_attr.T, We1.T, be1.reshape(ENC, 1),
                       Wec.T, bec.reshape(8, 1))
    eeP = [p.reshape(E // C, C) for p in eeP]
    srcI = srcI.reshape(E // C, C)
    dstI = dstI.reshape(E // C, C)
    partials = _sc_scatter(srcT, dstT, eeP, srcI, dstI, consts)
    actor, value = _finish(partials, ab, cb)
    return actor, value[0, 0]


# speedup vs baseline: 73.6718x; 1.1875x over previous
import jax
import jax.numpy as jnp
from jax import lax
from jax.experimental import pallas as pl
from jax.experimental.pallas import tpu as pltpu
from jax.experimental.pallas import tpu_sc as plsc

N = 10000
E = 320000
D_NODE = 128
D_EDGE = 16
ENC = 128
OUTS = 6

NC = 2
NS = 16
NW = NC * NS
EPW = E // NW
C = 800
TOT_CHUNKS = E // C
NCHUNK = -(-TOT_CHUNKS // NW)
G = C // 16
ZR8 = 1000

F32 = jnp.float32


def _nodes_body(x_ref, Wn1_ref, bn1_ref, Wn2_ref, bn2_ref, Wsrc_ref, Wdst_ref,
                srcT_ref, dstT_ref):
    h = jnp.maximum(x_ref[...] @ Wn1_ref[...] + bn1_ref[...], 0.0)
    h = h @ Wn2_ref[...] + bn2_ref[...]
    srcT_ref[...] = h @ Wsrc_ref[...]
    dstT_ref[...] = h @ Wdst_ref[...]


def _node_tables(x, Wn1, bn1, Wn2, bn2, Wsrc, Wdst):
    BN = 2000
    grid = (N // BN,)
    return pl.pallas_call(
        _nodes_body,
        grid=grid,
        in_specs=[
            pl.BlockSpec((BN, D_NODE), lambda i: (i, 0)),
            pl.BlockSpec((D_NODE, ENC), lambda i: (0, 0)),
            pl.BlockSpec((1, ENC), lambda i: (0, 0)),
            pl.BlockSpec((ENC, ENC), lambda i: (0, 0)),
            pl.BlockSpec((1, ENC), lambda i: (0, 0)),
            pl.BlockSpec((ENC, 16), lambda i: (0, 0)),
            pl.BlockSpec((ENC, 16), lambda i: (0, 0)),
        ],
        out_specs=[
            pl.BlockSpec((BN, 16), lambda i: (i, 0)),
            pl.BlockSpec((BN, 16), lambda i: (i, 0)),
        ],
        out_shape=[
            jax.ShapeDtypeStruct((N, 16), F32),
            jax.ShapeDtypeStruct((N, 16), F32),
        ],
    )(x, Wn1, bn1.reshape(1, ENC), Wn2, bn2.reshape(1, ENC), Wsrc, Wdst)


def _edges_body(eaT_ref, W1T_ref, b1c_ref, W2T_ref, b2c_ref, *out_refs):
    t = jnp.maximum(W1T_ref[...] @ eaT_ref[...] + b1c_ref[...], 0.0)
    ee = W2T_ref[...] @ t + b2c_ref[...]
    for j in range(7):
        out_refs[j][...] = ee[j].reshape(out_refs[j].shape)


def _edge_planes(eaT, W1T, b1c, W2T, b2c):
    BE = 16000
    grid = (E // BE,)
    return pl.pallas_call(
        _edges_body,
        grid=grid,
        in_specs=[
            pl.BlockSpec((D_EDGE, BE), lambda i: (0, i)),
            pl.BlockSpec((ENC, D_EDGE), lambda i: (0, 0)),
            pl.BlockSpec((ENC, 1), lambda i: (0, 0)),
            pl.BlockSpec((8, ENC), lambda i: (0, 0)),
            pl.BlockSpec((8, 1), lambda i: (0, 0)),
        ],
        out_specs=[pl.BlockSpec((1, BE // 128, 128), lambda i: (i, 0, 0))
                   for _ in range(7)],
        out_shape=[jax.ShapeDtypeStruct((E // BE, BE // 128, 128), F32)
                   for _ in range(7)],
    )(eaT, W1T, b1c, W2T, b2c)


def _sc_body(srcT_h, dstT_h, e0_h, e1_h, e2_h, e3_h, e4_h, e5_h, e6_h,
             srcI_h, dstI_h, consts_h, out_h,
             si0, di0, sr0, dr0, ee0, si1, di1, sr1, dr1, ee1,
             vals_v, cst_v, zbuf_v, acc_s, semI, semG):
    c = lax.axis_index("c")
    s = lax.axis_index("s")
    wid = c * NS + s
    ee_hs = (e0_h, e1_h, e2_h, e3_h, e4_h, e5_h, e6_h)

    pltpu.sync_copy(consts_h, cst_v)

    def _z(i, carry):
        zbuf_v[i] = jnp.zeros((16,), F32)
        return carry

    lax.fori_loop(0, 500, _z, 0, unroll=8)

    @pl.when(s < N // ZR8)
    def _zero_acc():
        pltpu.sync_copy(zbuf_v, acc_s.at[pl.ds(s * ZR8, 500)])
        pltpu.sync_copy(zbuf_v, acc_s.at[pl.ds(s * ZR8 + 500, 500)])

    plsc.subcore_barrier()

    attrow = cst_v[0]
    a_s = [attrow[j] for j in range(6)]
    catt_s = attrow[6]
    iota16 = lax.iota(jnp.int32, 16)
    col = [jnp.full((16,), j, jnp.int32) for j in range(16)]

    bufs = [(si0, di0, sr0, dr0, ee0), (si1, di1, sr1, dr1, ee1)]

    def issue_idx(k):
        si, di, _, _, ee = bufs[k % 2]
        cid = k * NW + wid
        cps = [pltpu.async_copy(srcI_h.at[cid], si, semI),
               pltpu.async_copy(dstI_h.at[cid], di, semI)]
        for j in range(7):
            cps.append(pltpu.async_copy(ee_hs[j].at[cid],
                                        ee.at[pl.ds(j * C, C)], semI))
        return cps

    def issue_main(k):
        si, di, sr, dr, _ = bufs[k % 2]
        return [pltpu.async_copy(srcT_h.at[si], sr, semG),
                pltpu.async_copy(dstT_h.at[di], dr, semG)]

    def compute_chunk(si, di, sr, dr, ee):
        def _group(g, carry):
            ridx = iota16 + g * 16
            sv = [plsc.load_gather(sr, [ridx, col[j]]) for j in range(7)]
            dv = [plsc.load_gather(dr, [ridx, col[j]]) for j in range(7)]
            l = []
            for j in range(7):
                u = sv[j] + dv[j] + ee[pl.ds(j * C + g * 16, 16)]
                l.append(jnp.maximum(u, 0.0) + 0.2 * jnp.minimum(u, 0.0))
            la = ((l[0] * a_s[0] + l[1] * a_s[1])
                  + (l[2] * a_s[2] + l[3] * a_s[3])
                  + (l[4] * a_s[4] + l[5] * a_s[5]))
            pa = jnp.exp(la)
            pc = jnp.exp(l[6] * catt_s)
            for j in range(6):
                plsc.store_scatter(vals_v, [ridx, col[8 + j]], pa * sv[j])
            plsc.store_scatter(vals_v, [ridx, col[14]], pa)
            plsc.store_scatter(vals_v, [ridx, col[6]], pc * sv[6])
            plsc.store_scatter(vals_v, [ridx, col[7]], pc)
            return carry

        lax.fori_loop(0, G, _group, 0, unroll=2)
        pltpu.sync_copy(vals_v, acc_s.at[di], add=True)

    NSTAT = TOT_CHUNKS // NW
    idx_cps = {0: issue_idx(0)}
    for cp in idx_cps[0]:
        cp.wait()
    main_cps = {0: issue_main(0)}
    idx_cps[1] = issue_idx(1)

    for k in range(NSTAT):
        si, di, sr, dr, ee = bufs[k % 2]
        for cp in main_cps[k]:
            cp.wait()
        if k + 1 < NSTAT:
            for cp in idx_cps[k + 1]:
                cp.wait()
            main_cps[k + 1] = issue_main(k + 1)
        compute_chunk(si, di, sr, dr, ee)
        if k + 2 < NSTAT:
            idx_cps[k + 2] = issue_idx(k + 2)

    TAIL = TOT_CHUNKS - NSTAT * NW
    if TAIL:
        @pl.when(wid < TAIL)
        def _tail():
            si, di, sr, dr, ee = bufs[NSTAT % 2]
            for cp in issue_idx(NSTAT):
                cp.wait()
            for cp in issue_main(NSTAT):
                cp.wait()
            compute_chunk(si, di, sr, dr, ee)

    plsc.subcore_barrier()

    @pl.when(s < N // ZR8)
    def _dump_acc():
        pltpu.sync_copy(acc_s.at[pl.ds(s * ZR8, ZR8)],
                        out_h.at[pl.ds(c * N + s * ZR8, ZR8)])


def _sc_scatter(srcT, dstT, eeP, srcI, dstI, consts):
    mesh = plsc.VectorSubcoreMesh(core_axis_name="c", subcore_axis_name="s",
                                  num_cores=NC, num_subcores=NS)
    fn = pl.kernel(
        _sc_body,
        out_type=jax.ShapeDtypeStruct((NC * N, 16), F32),
        mesh=mesh,
        compiler_params=pltpu.CompilerParams(needs_layout_passes=False,
                                             use_tc_tiling_on_sc=False),
        scratch_types=[
            pltpu.VMEM((C,), jnp.int32),
            pltpu.VMEM((C,), jnp.int32),
            pltpu.VMEM((C, 16), F32),
            pltpu.VMEM((C, 16), F32),
            pltpu.VMEM((7 * C,), F32),
            pltpu.VMEM((C,), jnp.int32),
            pltpu.VMEM((C,), jnp.int32),
            pltpu.VMEM((C, 16), F32),
            pltpu.VMEM((C, 16), F32),
            pltpu.VMEM((7 * C,), F32),
            pltpu.VMEM((C, 16), F32),
            pltpu.VMEM((8, 16), F32),
            pltpu.VMEM((500, 16), F32),
            pltpu.VMEM_SHARED((N, 16), F32),
            pltpu.SemaphoreType.DMA,
            pltpu.SemaphoreType.DMA,
        ],
    )
    return fn(srcT, dstT, *eeP, srcI, dstI, consts)


def _finish_body(p_ref, ab_ref, cb_ref, actor_ref, value_ref):
    acc = p_ref[pl.ds(0, N), :] + p_ref[pl.ds(N, N), :]
    actor_ref[...] = acc[:, 8:14] / (acc[:, 14:15] + 1e-16) + ab_ref[...]
    critic = acc[:, 6] / (acc[:, 7] + 1e-16) + cb_ref[0, 0]
    value_ref[...] = jnp.broadcast_to(jnp.sum(critic) * (1.0 / N), (1, 128))


def _finish(partials, ab, cb):
    return pl.pallas_call(
        _finish_body,
        out_shape=[
            jax.ShapeDtypeStruct((N, OUTS), F32),
            jax.ShapeDtypeStruct((1, 128), F32),
        ],
    )(partials, ab.reshape(1, OUTS), cb.reshape(1, 1))


def kernel(x, edge_index, edge_attr, Wn1, bn1, Wn2, bn2, We1, be1, We2, be2,
           aWl, aWr, aWe, aatt, ab, cWl, cWr, cWe, catt, cb):
    z9 = jnp.zeros((ENC, 9), F32)
    Wsrc = jnp.concatenate([aWl, cWl, z9], axis=1)
    Wdst = jnp.concatenate([aWr, cWr, z9], axis=1)
    Wcat = jnp.concatenate([aWe, cWe, jnp.zeros((ENC, 1), F32)], axis=1)
    Wec = jnp.dot(We2, Wcat, precision=lax.Precision.HIGHEST)
    bec = jnp.dot(be2, Wcat, precision=lax.Precision.HIGHEST)

    consts = jnp.zeros((8, 16), F32)
    consts = consts.at[0, 0:6].set(aatt)
    consts = consts.at[0, 6].set(catt[0])

    srcI = edge_index[0].astype(jnp.int32)
    dstI = edge_index[1].astype(jnp.int32)

    srcT, dstT = _node_tables(x, Wn1, bn1, Wn2, bn2, Wsrc, Wdst)
    eeP = _edge_planes(edge_attr.T, We1.T, be1.reshape(ENC, 1),
                       Wec.T, bec.reshape(8, 1))
    eeP = [p.reshape(E // C, C) for p in eeP]
    srcI = srcI.reshape(E // C, C)
    dstI = dstI.reshape(E // C, C)
    partials = _sc_scatter(srcT, dstT, eeP, srcI, dstI, consts)
    actor, value = _finish(partials, ab, cb)
    return actor, value[0, 0]


# speedup vs baseline: 73.9617x; 1.0039x over previous
import jax
import jax.numpy as jnp
from jax import lax
from jax.experimental import pallas as pl
from jax.experimental.pallas import tpu as pltpu
from jax.experimental.pallas import tpu_sc as plsc

N = 10000
E = 320000
D_NODE = 128
D_EDGE = 16
ENC = 128
OUTS = 6

NC = 2
NS = 16
NW = NC * NS
EPW = E // NW
C = 800
TOT_CHUNKS = E // C
NCHUNK = -(-TOT_CHUNKS // NW)
G = C // 16
ZR8 = 1000

F32 = jnp.float32


def _nodes_body(x_ref, Wn1_ref, bn1_ref, Wn2_ref, bn2_ref, Wsrc_ref, Wdst_ref,
                srcT_ref, dstT_ref):
    h = jnp.maximum(x_ref[...] @ Wn1_ref[...] + bn1_ref[...], 0.0)
    h = h @ Wn2_ref[...] + bn2_ref[...]
    srcT_ref[...] = h @ Wsrc_ref[...]
    dstT_ref[...] = h @ Wdst_ref[...]


def _node_tables(x, Wn1, bn1, Wn2, bn2, Wsrc, Wdst):
    BN = 2000
    grid = (N // BN,)
    return pl.pallas_call(
        _nodes_body,
        grid=grid,
        in_specs=[
            pl.BlockSpec((BN, D_NODE), lambda i: (i, 0)),
            pl.BlockSpec((D_NODE, ENC), lambda i: (0, 0)),
            pl.BlockSpec((1, ENC), lambda i: (0, 0)),
            pl.BlockSpec((ENC, ENC), lambda i: (0, 0)),
            pl.BlockSpec((1, ENC), lambda i: (0, 0)),
            pl.BlockSpec((ENC, 16), lambda i: (0, 0)),
            pl.BlockSpec((ENC, 16), lambda i: (0, 0)),
        ],
        out_specs=[
            pl.BlockSpec((BN, 16), lambda i: (i, 0)),
            pl.BlockSpec((BN, 16), lambda i: (i, 0)),
        ],
        out_shape=[
            jax.ShapeDtypeStruct((N, 16), F32),
            jax.ShapeDtypeStruct((N, 16), F32),
        ],
    )(x, Wn1, bn1.reshape(1, ENC), Wn2, bn2.reshape(1, ENC), Wsrc, Wdst)


def _edges_body(eaT_ref, W1T_ref, b1c_ref, W2T_ref, b2c_ref, *out_refs):
    t = jnp.maximum(W1T_ref[...] @ eaT_ref[...] + b1c_ref[...], 0.0)
    ee = W2T_ref[...] @ t + b2c_ref[...]
    for j in range(7):
        out_refs[j][...] = ee[j].reshape(out_refs[j].shape)


def _edge_planes(eaT, W1T, b1c, W2T, b2c):
    BE = 16000
    grid = (E // BE,)
    return pl.pallas_call(
        _edges_body,
        grid=grid,
        in_specs=[
            pl.BlockSpec((D_EDGE, BE), lambda i: (0, i)),
            pl.BlockSpec((ENC, D_EDGE), lambda i: (0, 0)),
            pl.BlockSpec((ENC, 1), lambda i: (0, 0)),
            pl.BlockSpec((8, ENC), lambda i: (0, 0)),
            pl.BlockSpec((8, 1), lambda i: (0, 0)),
        ],
        out_specs=[pl.BlockSpec((1, BE // 128, 128), lambda i: (i, 0, 0))
                   for _ in range(7)],
        out_shape=[jax.ShapeDtypeStruct((E // BE, BE // 128, 128), F32)
                   for _ in range(7)],
    )(eaT, W1T, b1c, W2T, b2c)


def _sc_body(srcT_h, dstT_h, e0_h, e1_h, e2_h, e3_h, e4_h, e5_h, e6_h,
             srcI_h, dstI_h, consts_h, out_h,
             si0, di0, sr0, dr0, ee0, si1, di1, sr1, dr1, ee1,
             vals_v, cst_v, zbuf_v, acc_s, srcS_s, dstS_s, semI, semG):
    c = lax.axis_index("c")
    s = lax.axis_index("s")
    wid = c * NS + s
    ee_hs = (e0_h, e1_h, e2_h, e3_h, e4_h, e5_h, e6_h)

    pltpu.sync_copy(consts_h, cst_v)

    def _z(i, carry):
        zbuf_v[i] = jnp.zeros((16,), F32)
        return carry

    lax.fori_loop(0, 500, _z, 0, unroll=8)

    @pl.when(s < N // ZR8)
    def _zero_acc():
        pltpu.sync_copy(zbuf_v, acc_s.at[pl.ds(s * ZR8, 500)])
        pltpu.sync_copy(zbuf_v, acc_s.at[pl.ds(s * ZR8 + 500, 500)])

    @pl.when(s < N // ZR8)
    def _stage_tables():
        pltpu.sync_copy(srcT_h.at[pl.ds(s * ZR8, ZR8)],
                        srcS_s.at[pl.ds(s * ZR8, ZR8)])
        pltpu.sync_copy(dstT_h.at[pl.ds(s * ZR8, ZR8)],
                        dstS_s.at[pl.ds(s * ZR8, ZR8)])

    plsc.subcore_barrier()

    attrow = cst_v[0]
    a_s = [attrow[j] for j in range(6)]
    catt_s = attrow[6]
    iota16 = lax.iota(jnp.int32, 16)
    col = [jnp.full((16,), j, jnp.int32) for j in range(16)]

    bufs = [(si0, di0, sr0, dr0, ee0), (si1, di1, sr1, dr1, ee1)]

    def issue_idx(k):
        si, di, _, _, ee = bufs[k % 2]
        cid = k * NW + wid
        cps = [pltpu.async_copy(srcI_h.at[cid], si, semI),
               pltpu.async_copy(dstI_h.at[cid], di, semI)]
        for j in range(7):
            cps.append(pltpu.async_copy(ee_hs[j].at[cid],
                                        ee.at[pl.ds(j * C, C)], semI))
        return cps

    def issue_main(k):
        si, di, sr, dr, _ = bufs[k % 2]
        return [pltpu.async_copy(srcS_s.at[si], sr, semG),
                pltpu.async_copy(dstS_s.at[di], dr, semG)]

    def compute_chunk(si, di, sr, dr, ee):
        def _group(g, carry):
            ridx = iota16 + g * 16
            sv = [plsc.load_gather(sr, [ridx, col[j]]) for j in range(7)]
            dv = [plsc.load_gather(dr, [ridx, col[j]]) for j in range(7)]
            l = []
            for j in range(7):
                u = sv[j] + dv[j] + ee[pl.ds(j * C + g * 16, 16)]
                l.append(jnp.maximum(u, 0.0) + 0.2 * jnp.minimum(u, 0.0))
            la = ((l[0] * a_s[0] + l[1] * a_s[1])
                  + (l[2] * a_s[2] + l[3] * a_s[3])
                  + (l[4] * a_s[4] + l[5] * a_s[5]))
            pa = jnp.exp(la)
            pc = jnp.exp(l[6] * catt_s)
            for j in range(6):
                plsc.store_scatter(vals_v, [ridx, col[8 + j]], pa * sv[j])
            plsc.store_scatter(vals_v, [ridx, col[14]], pa)
            plsc.store_scatter(vals_v, [ridx, col[6]], pc * sv[6])
            plsc.store_scatter(vals_v, [ridx, col[7]], pc)
            return carry

        lax.fori_loop(0, G, _group, 0, unroll=2)
        pltpu.sync_copy(vals_v, acc_s.at[di], add=True)

    NSTAT = TOT_CHUNKS // NW
    idx_cps = {0: issue_idx(0)}
    for cp in idx_cps[0]:
        cp.wait()
    main_cps = {0: issue_main(0)}
    idx_cps[1] = issue_idx(1)

    for k in range(NSTAT):
        si, di, sr, dr, ee = bufs[k % 2]
        for cp in main_cps[k]:
            cp.wait()
        if k + 1 < NSTAT:
            for cp in idx_cps[k + 1]:
                cp.wait()
            main_cps[k + 1] = issue_main(k + 1)
        compute_chunk(si, di, sr, dr, ee)
        if k + 2 < NSTAT:
            idx_cps[k + 2] = issue_idx(k + 2)

    TAIL = TOT_CHUNKS - NSTAT * NW
    if TAIL:
        @pl.when(wid < TAIL)
        def _tail():
            si, di, sr, dr, ee = bufs[NSTAT % 2]
            for cp in issue_idx(NSTAT):
                cp.wait()
            for cp in issue_main(NSTAT):
                cp.wait()
            compute_chunk(si, di, sr, dr, ee)

    plsc.subcore_barrier()

    @pl.when(s < N // ZR8)
    def _dump_acc():
        pltpu.sync_copy(acc_s.at[pl.ds(s * ZR8, ZR8)],
                        out_h.at[pl.ds(c * N + s * ZR8, ZR8)])


def _sc_scatter(srcT, dstT, eeP, srcI, dstI, consts):
    mesh = plsc.VectorSubcoreMesh(core_axis_name="c", subcore_axis_name="s",
                                  num_cores=NC, num_subcores=NS)
    fn = pl.kernel(
        _sc_body,
        out_type=jax.ShapeDtypeStruct((NC * N, 16), F32),
        mesh=mesh,
        compiler_params=pltpu.CompilerParams(needs_layout_passes=False,
                                             use_tc_tiling_on_sc=False),
        scratch_types=[
            pltpu.VMEM((C,), jnp.int32),
            pltpu.VMEM((C,), jnp.int32),
            pltpu.VMEM((C, 16), F32),
            pltpu.VMEM((C, 16), F32),
            pltpu.VMEM((7 * C,), F32),
            pltpu.VMEM((C,), jnp.int32),
            pltpu.VMEM((C,), jnp.int32),
            pltpu.VMEM((C, 16), F32),
            pltpu.VMEM((C, 16), F32),
            pltpu.VMEM((7 * C,), F32),
            pltpu.VMEM((C, 16), F32),
            pltpu.VMEM((8, 16), F32),
            pltpu.VMEM((500, 16), F32),
            pltpu.VMEM_SHARED((N, 16), F32),
            pltpu.VMEM_SHARED((N, 16), F32),
            pltpu.VMEM_SHARED((N, 16), F32),
            pltpu.SemaphoreType.DMA,
            pltpu.SemaphoreType.DMA,
        ],
    )
    return fn(srcT, dstT, *eeP, srcI, dstI, consts)


def _finish_body(p_ref, ab_ref, cb_ref, actor_ref, value_ref):
    acc = p_ref[pl.ds(0, N), :] + p_ref[pl.ds(N, N), :]
    actor_ref[...] = acc[:, 8:14] / (acc[:, 14:15] + 1e-16) + ab_ref[...]
    critic = acc[:, 6] / (acc[:, 7] + 1e-16) + cb_ref[0, 0]
    value_ref[...] = jnp.broadcast_to(jnp.sum(critic) * (1.0 / N), (1, 128))


def _finish(partials, ab, cb):
    return pl.pallas_call(
        _finish_body,
        out_shape=[
            jax.ShapeDtypeStruct((N, OUTS), F32),
            jax.ShapeDtypeStruct((1, 128), F32),
        ],
    )(partials, ab.reshape(1, OUTS), cb.reshape(1, 1))


def kernel(x, edge_index, edge_attr, Wn1, bn1, Wn2, bn2, We1, be1, We2, be2,
           aWl, aWr, aWe, aatt, ab, cWl, cWr, cWe, catt, cb):
    z9 = jnp.zeros((ENC, 9), F32)
    Wsrc = jnp.concatenate([aWl, cWl, z9], axis=1)
    Wdst = jnp.concatenate([aWr, cWr, z9], axis=1)
    Wcat = jnp.concatenate([aWe, cWe, jnp.zeros((ENC, 1), F32)], axis=1)
    Wec = jnp.dot(We2, Wcat, precision=lax.Precision.HIGHEST)
    bec = jnp.dot(be2, Wcat, precision=lax.Precision.HIGHEST)

    consts = jnp.zeros((8, 16), F32)
    consts = consts.at[0, 0:6].set(aatt)
    consts = consts.at[0, 6].set(catt[0])

    srcI = edge_index[0].astype(jnp.int32)
    dstI = edge_index[1].astype(jnp.int32)

    srcT, dstT = _node_tables(x, Wn1, bn1, Wn2, bn2, Wsrc, Wdst)
    eeP = _edge_planes(edge_attr.T, We1.T, be1.reshape(ENC, 1),
                       Wec.T, bec.reshape(8, 1))
    eeP = [p.reshape(E // C, C) for p in eeP]
    srcI = srcI.reshape(E // C, C)
    dstI = dstI.reshape(E // C, C)
    partials = _sc_scatter(srcT, dstT, eeP, srcI, dstI, consts)
    actor, value = _finish(partials, ab, cb)
    return actor, value[0, 0]


# speedup vs baseline: 75.7795x; 1.0246x over previous
import jax
import jax.numpy as jnp
from jax import lax
from jax.experimental import pallas as pl
from jax.experimental.pallas import tpu as pltpu
from jax.experimental.pallas import tpu_sc as plsc

N = 10000
E = 320000
D_NODE = 128
D_EDGE = 16
ENC = 128
OUTS = 6

NC = 2
NS = 16
NW = NC * NS
EPW = E // NW
C = 800
TOT_CHUNKS = E // C
NCHUNK = -(-TOT_CHUNKS // NW)
G = C // 16
ZR8 = 1000

F32 = jnp.float32


def _nodes_body(x_ref, Wn1_ref, bn1_ref, Wn2_ref, bn2_ref, Wsrc_ref, Wdst_ref,
                srcT_ref, dstT_ref):
    h = jnp.maximum(x_ref[...] @ Wn1_ref[...] + bn1_ref[...], 0.0)
    h = h @ Wn2_ref[...] + bn2_ref[...]
    srcT_ref[...] = h @ Wsrc_ref[...]
    dstT_ref[...] = h @ Wdst_ref[...]


def _node_tables(x, Wn1, bn1, Wn2, bn2, Wsrc, Wdst):
    BN = 2000
    grid = (N // BN,)
    return pl.pallas_call(
        _nodes_body,
        grid=grid,
        in_specs=[
            pl.BlockSpec((BN, D_NODE), lambda i: (i, 0)),
            pl.BlockSpec((D_NODE, ENC), lambda i: (0, 0)),
            pl.BlockSpec((1, ENC), lambda i: (0, 0)),
            pl.BlockSpec((ENC, ENC), lambda i: (0, 0)),
            pl.BlockSpec((1, ENC), lambda i: (0, 0)),
            pl.BlockSpec((ENC, 16), lambda i: (0, 0)),
            pl.BlockSpec((ENC, 16), lambda i: (0, 0)),
        ],
        out_specs=[
            pl.BlockSpec((BN, 16), lambda i: (i, 0)),
            pl.BlockSpec((BN, 16), lambda i: (i, 0)),
        ],
        out_shape=[
            jax.ShapeDtypeStruct((N, 16), F32),
            jax.ShapeDtypeStruct((N, 16), F32),
        ],
    )(x, Wn1, bn1.reshape(1, ENC), Wn2, bn2.reshape(1, ENC), Wsrc, Wdst)


def _edges_body(eaT_ref, W1T_ref, b1c_ref, W2T_ref, b2c_ref, *out_refs):
    t = jnp.maximum(W1T_ref[...] @ eaT_ref[...] + b1c_ref[...], 0.0)
    ee = W2T_ref[...] @ t + b2c_ref[...]
    for j in range(7):
        out_refs[j][...] = ee[j].reshape(out_refs[j].shape)


def _edge_planes(eaT, W1T, b1c, W2T, b2c):
    BE = 16000
    grid = (E // BE,)
    return pl.pallas_call(
        _edges_body,
        grid=grid,
        in_specs=[
            pl.BlockSpec((D_EDGE, BE), lambda i: (0, i)),
            pl.BlockSpec((ENC, D_EDGE), lambda i: (0, 0)),
            pl.BlockSpec((ENC, 1), lambda i: (0, 0)),
            pl.BlockSpec((8, ENC), lambda i: (0, 0)),
            pl.BlockSpec((8, 1), lambda i: (0, 0)),
        ],
        out_specs=[pl.BlockSpec((1, BE // 128, 128), lambda i: (i, 0, 0))
                   for _ in range(7)],
        out_shape=[jax.ShapeDtypeStruct((E // BE, BE // 128, 128), F32)
                   for _ in range(7)],
    )(eaT, W1T, b1c, W2T, b2c)


def _sc_body(srcT_h, dstT_h, e0_h, e1_h, e2_h, e3_h, e4_h, e5_h, e6_h,
             srcI_h, dstI_h, consts_h, out_h,
             si0, di0, si1, di1, si2, di2, si3, di3,
             sr0, dr0, ee0, sr1, dr1, ee1,
             vals0, vals1, vals2, vals3,
             cst_v, acc_s, semI, semG, semS):
    c = lax.axis_index("c")
    s = lax.axis_index("s")
    wid = c * NS + s
    ee_hs = (e0_h, e1_h, e2_h, e3_h, e4_h, e5_h, e6_h)

    pltpu.sync_copy(consts_h, cst_v)

    def _z(i, carry):
        vals0[i] = jnp.zeros((16,), F32)
        return carry

    lax.fori_loop(0, 500, _z, 0, unroll=8)

    @pl.when(s < N // ZR8)
    def _zero_acc():
        pltpu.sync_copy(vals0.at[pl.ds(0, 500)], acc_s.at[pl.ds(s * ZR8, 500)])
        pltpu.sync_copy(vals0.at[pl.ds(0, 500)],
                        acc_s.at[pl.ds(s * ZR8 + 500, 500)])

    plsc.subcore_barrier()

    attrow = cst_v[0]
    a_s = [attrow[j] for j in range(6)]
    catt_s = attrow[6]
    iota16 = lax.iota(jnp.int32, 16)
    col = [jnp.full((16,), j, jnp.int32) for j in range(16)]

    idx4 = [(si0, di0), (si1, di1), (si2, di2), (si3, di3)]
    buf2 = [(sr0, dr0, ee0), (sr1, dr1, ee1)]
    vals4 = [vals0, vals1, vals2, vals3]

    def issue_idx(k):
        si, di = idx4[k % 4]
        ee = buf2[k % 2][2]
        cid = k * NW + wid
        cps = [pltpu.async_copy(srcI_h.at[cid], si, semI),
               pltpu.async_copy(dstI_h.at[cid], di, semI)]
        for j in range(7):
            cps.append(pltpu.async_copy(ee_hs[j].at[cid],
                                        ee.at[pl.ds(j * C, C)], semI))
        return cps

    def issue_main(k):
        si, di = idx4[k % 4]
        sr, dr, _ = buf2[k % 2]
        return [pltpu.async_copy(srcT_h.at[si], sr, semG),
                pltpu.async_copy(dstT_h.at[di], dr, semG)]

    def compute_chunk(sr, dr, ee, vals_v):
        def _group(g, carry):
            ridx = iota16 + g * 16
            sv = [plsc.load_gather(sr, [ridx, col[j]]) for j in range(7)]
            dv = [plsc.load_gather(dr, [ridx, col[j]]) for j in range(7)]
            l = []
            for j in range(7):
                u = sv[j] + dv[j] + ee[pl.ds(j * C + g * 16, 16)]
                l.append(jnp.maximum(u, 0.0) + 0.2 * jnp.minimum(u, 0.0))
            la = ((l[0] * a_s[0] + l[1] * a_s[1])
                  + (l[2] * a_s[2] + l[3] * a_s[3])
                  + (l[4] * a_s[4] + l[5] * a_s[5]))
            pa = jnp.exp(la)
            pc = jnp.exp(l[6] * catt_s)
            for j in range(6):
                plsc.store_scatter(vals_v, [ridx, col[8 + j]], pa * sv[j])
            plsc.store_scatter(vals_v, [ridx, col[14]], pa)
            plsc.store_scatter(vals_v, [ridx, col[6]], pc * sv[6])
            plsc.store_scatter(vals_v, [ridx, col[7]], pc)
            return carry

        lax.fori_loop(0, G, _group, 0, unroll=2)

    def issue_scatter(k):
        return pltpu.async_copy(vals4[k % 4], acc_s.at[idx4[k % 4][1]],
                                semS, add=True)

    NSTAT = TOT_CHUNKS // NW
    idx_cps = {0: issue_idx(0)}
    for cp in idx_cps[0]:
        cp.wait()
    main_cps = {0: issue_main(0)}
    idx_cps[1] = issue_idx(1)
    scat_cps = {}

    for k in range(NSTAT):
        sr, dr, ee = buf2[k % 2]
        for cp in main_cps[k]:
            cp.wait()
        if k + 1 < NSTAT:
            for cp in idx_cps[k + 1]:
                cp.wait()
            main_cps[k + 1] = issue_main(k + 1)
        compute_chunk(sr, dr, ee, vals4[k % 4])
        scat_cps[k] = issue_scatter(k)
        if k + 2 < NSTAT:
            if k - 2 >= 0:
                scat_cps[k - 2].wait()
            idx_cps[k + 2] = issue_idx(k + 2)

    for k in range(max(0, NSTAT - 4), NSTAT):
        scat_cps[k].wait()

    TAIL = TOT_CHUNKS - NSTAT * NW
    if TAIL:
        @pl.when(wid < TAIL)
        def _tail():
            sr, dr, ee = buf2[NSTAT % 2]
            for cp in issue_idx(NSTAT):
                cp.wait()
            for cp in issue_main(NSTAT):
                cp.wait()
            compute_chunk(sr, dr, ee, vals4[NSTAT % 4])
            pltpu.async_copy(vals4[NSTAT % 4],
                             acc_s.at[idx4[NSTAT % 4][1]],
                             semS, add=True).wait()

    plsc.subcore_barrier()

    @pl.when(s < N // ZR8)
    def _dump_acc():
        pltpu.sync_copy(acc_s.at[pl.ds(s * ZR8, ZR8)],
                        out_h.at[pl.ds(c * N + s * ZR8, ZR8)])


def _sc_scatter(srcT, dstT, eeP, srcI, dstI, consts):
    mesh = plsc.VectorSubcoreMesh(core_axis_name="c", subcore_axis_name="s",
                                  num_cores=NC, num_subcores=NS)
    fn = pl.kernel(
        _sc_body,
        out_type=jax.ShapeDtypeStruct((NC * N, 16), F32),
        mesh=mesh,
        compiler_params=pltpu.CompilerParams(needs_layout_passes=False,
                                             use_tc_tiling_on_sc=False),
        scratch_types=(
            [pltpu.VMEM((C,), jnp.int32) for _ in range(8)]
            + [pltpu.VMEM((C, 16), F32), pltpu.VMEM((C, 16), F32),
               pltpu.VMEM((7 * C,), F32)] * 2
            + [pltpu.VMEM((C, 16), F32) for _ in range(4)]
            + [pltpu.VMEM((8, 16), F32)]
            + [pltpu.VMEM_SHARED((N, 16), F32)]
            + [pltpu.SemaphoreType.DMA] * 3
        ),
    )
    return fn(srcT, dstT, *eeP, srcI, dstI, consts)


def _finish_body(p_ref, ab_ref, cb_ref, actor_ref, value_ref):
    acc = p_ref[pl.ds(0, N), :] + p_ref[pl.ds(N, N), :]
    actor_ref[...] = acc[:, 8:14] / (acc[:, 14:15] + 1e-16) + ab_ref[...]
    critic = acc[:, 6] / (acc[:, 7] + 1e-16) + cb_ref[0, 0]
    value_ref[...] = jnp.broadcast_to(jnp.sum(critic) * (1.0 / N), (1, 128))


def _finish(partials, ab, cb):
    return pl.pallas_call(
        _finish_body,
        out_shape=[
            jax.ShapeDtypeStruct((N, OUTS), F32),
            jax.ShapeDtypeStruct((1, 128), F32),
        ],
    )(partials, ab.reshape(1, OUTS), cb.reshape(1, 1))


def kernel(x, edge_index, edge_attr, Wn1, bn1, Wn2, bn2, We1, be1, We2, be2,
           aWl, aWr, aWe, aatt, ab, cWl, cWr, cWe, catt, cb):
    z9 = jnp.zeros((ENC, 9), F32)
    Wsrc = jnp.concatenate([aWl, cWl, z9], axis=1)
    Wdst = jnp.concatenate([aWr, cWr, z9], axis=1)
    Wcat = jnp.concatenate([aWe, cWe, jnp.zeros((ENC, 1), F32)], axis=1)
    Wec = jnp.dot(We2, Wcat, precision=lax.Precision.HIGHEST)
    bec = jnp.dot(be2, Wcat, precision=lax.Precision.HIGHEST)

    consts = jnp.zeros((8, 16), F32)
    consts = consts.at[0, 0:6].set(aatt)
    consts = consts.at[0, 6].set(catt[0])

    srcI = edge_index[0].astype(jnp.int32)
    dstI = edge_index[1].astype(jnp.int32)

    srcT, dstT = _node_tables(x, Wn1, bn1, Wn2, bn2, Wsrc, Wdst)
    eeP = _edge_planes(edge_attr.T, We1.T, be1.reshape(ENC, 1),
                       Wec.T, bec.reshape(8, 1))
    eeP = [p.reshape(E // C, C) for p in eeP]
    srcI = srcI.reshape(E // C, C)
    dstI = dstI.reshape(E // C, C)
    partials = _sc_scatter(srcT, dstT, eeP, srcI, dstI, consts)
    actor, value = _finish(partials, ab, cb)
    return actor, value[0, 0]
